# baseline (speedup 1.0000x reference)
import functools

import jax
import jax.numpy as jnp
from jax import lax
from jax.experimental import pallas as pl
from jax.experimental.pallas import tpu as pltpu
from jax.experimental.pallas import tpu_sc as plsc

N = 10000
C = 128
E = 320000
HALF = N // 2
ACC_R = 5200
DUMMY = HALF
B = 640
NB = E // B
GB = 128
RB = 200
C2 = 2 * C


def _sc_spmm(x_flat, meta, n_tiles_arg=None):
    mesh = plsc.VectorSubcoreMesh(core_axis_name="c", subcore_axis_name="s")

    @functools.partial(
        pl.kernel,
        mesh=mesh,
        compiler_params=pltpu.CompilerParams(
            needs_layout_passes=False, use_tc_tiling_on_sc=False),
        out_type=jax.ShapeDtypeStruct((2, C2, ACC_R), jnp.float32),
        scratch_types=[
            pltpu.VMEM((B, 4), jnp.int32),
            pltpu.VMEM((B,), jnp.int32),
            pltpu.VMEM((B, 16), jnp.float32),
            pltpu.VMEM((16, ACC_R), jnp.float32),
        ],
    )
    def k(x_hbm, meta_hbm, a_out, meta_v, colg_v, xbuf, acc_t):
        c = lax.axis_index("c")
        s = lax.axis_index("s")
        row_base = c * HALF
        zero16 = jnp.zeros((16,), jnp.float32)
        iota = lax.iota(jnp.int32, 16)

        def zrow(r, carry):
            for ch in range(16):
                acc_t[ch, pl.ds(r * 16, 16)] = zero16
            return carry
        lax.fori_loop(0, ACC_R // 16, zrow, 0)

        def blk_body(b, carry):
            e0 = b * B
            pltpu.sync_copy(meta_hbm.at[pl.ds(e0, B), :], meta_v)

            def prep(kk, carry2):
                eidx = kk * 16 + iota
                cols16 = plsc.load_gather(meta_v, [eidx, jnp.full((16,), 1, jnp.int32)])
                colg_v[pl.ds(kk * 16, 16)] = cols16 * 16 + s
                return carry2
            lax.fori_loop(0, B // 16, prep, 0)

            for q in range(B // GB):
                pltpu.sync_copy(x_hbm.at[colg_v.at[pl.ds(q * GB, GB)]],
                                xbuf.at[pl.ds(q * GB, GB)])

            def grp_body(kk, carry2):
                eidx = kk * 16 + iota
                rows16 = plsc.load_gather(meta_v, [eidx, jnp.full((16,), 0, jnp.int32)])
                vrr = plsc.bitcast(
                    plsc.load_gather(meta_v, [eidx, jnp.full((16,), 2, jnp.int32)]),
                    jnp.float32)
                vii = plsc.bitcast(
                    plsc.load_gather(meta_v, [eidx, jnp.full((16,), 3, jnp.int32)]),
                    jnp.float32)
                local = rows16 - row_base
                ok = (local >= 0) & (local < HALF)
                lr = jnp.where(ok, local, DUMMY)
                for m in range(8):
                    xr = plsc.load_gather(xbuf, [eidx, jnp.full((16,), 2 * m, jnp.int32)])
                    xi = plsc.load_gather(xbuf, [eidx, jnp.full((16,), 2 * m + 1, jnp.int32)])
                    orv = vrr * xr - vii * xi
                    oiv = vii * xr + vrr * xi
                    plsc.addupdate_scatter(
                        acc_t, [jnp.full((16,), 2 * m, jnp.int32), lr], orv)
                    plsc.addupdate_scatter(
                        acc_t, [jnp.full((16,), 2 * m + 1, jnp.int32), lr], oiv)
                return carry2
            lax.fori_loop(0, B // 16, grp_body, 0)
            return carry
        lax.fori_loop(0, NB, blk_body, 0)

        pltpu.sync_copy(acc_t, a_out.at[c, pl.ds(16 * s, 16), :])

    return k(x_flat, meta)


def _tc_body(a_ref, w_ref, xr_ref, xi_ref, or_ref, oi_ref):
    a = a_ref[0]
    res = lax.dot_general(a, w_ref[...], (((0,), (0,)), ((), ())),
                          preferred_element_type=jnp.float32)
    or_ref[...] = res[:HALF, :C] + xr_ref[...]
    oi_ref[...] = res[:HALF, C:] + xi_ref[...]


def _tc_matmul(a_full, x_real, x_imag, w_big):
    return pl.pallas_call(
        _tc_body,
        grid=(2,),
        in_specs=[
            pl.BlockSpec((1, C2, ACC_R), lambda i: (i, 0, 0)),
            pl.BlockSpec((C2, C2), lambda i: (0, 0)),
            pl.BlockSpec((HALF, C), lambda i: (i, 0)),
            pl.BlockSpec((HALF, C), lambda i: (i, 0)),
        ],
        out_specs=[
            pl.BlockSpec((HALF, C), lambda i: (i, 0)),
            pl.BlockSpec((HALF, C), lambda i: (i, 0)),
        ],
        out_shape=[
            jax.ShapeDtypeStruct((N, C), jnp.float32),
            jax.ShapeDtypeStruct((N, C), jnp.float32),
        ],
    )(a_full, w_big, x_real, x_imag)


@jax.jit
def kernel(X_real, X_imag, edge_index, L_real_vals, L_imag_vals, weight):
    x_flat = jnp.stack([X_real, X_imag], axis=2).reshape(N * 16, 16)
    meta = jnp.stack(
        [edge_index[0], edge_index[1],
         jax.lax.bitcast_convert_type(L_real_vals, jnp.int32),
         jax.lax.bitcast_convert_type(L_imag_vals, jnp.int32)], axis=1)
    ch = jnp.arange(C)
    u_r = (ch // 8) * 16 + (ch % 8) * 2
    w_big = jnp.zeros((C2, C2), jnp.float32)
    w_big = w_big.at[u_r, :C].set(weight).at[u_r + 1, C:].set(weight)

    a_full = _sc_spmm(x_flat, meta)
    return _tc_matmul(a_full, X_real, X_imag, w_big)


# speedup vs baseline: 1.2638x; 1.2638x over previous
import functools

import jax
import jax.numpy as jnp
from jax import lax
from jax.experimental import pallas as pl
from jax.experimental.pallas import tpu as pltpu
from jax.experimental.pallas import tpu_sc as plsc

N = 10000
C = 128
E = 320000
HALF = N // 2
ACC_R = 5200
DUMMY = HALF
B = 640
NB = E // B
GB = 128
RB = 200
C2 = 2 * C


def _sc_spmm(x_flat, meta, n_tiles_arg=None):
    mesh = plsc.VectorSubcoreMesh(core_axis_name="c", subcore_axis_name="s")

    @functools.partial(
        pl.kernel,
        mesh=mesh,
        compiler_params=pltpu.CompilerParams(
            needs_layout_passes=False, use_tc_tiling_on_sc=False),
        out_type=jax.ShapeDtypeStruct((2, C2, ACC_R), jnp.float32),
        scratch_types=[
            pltpu.VMEM((B, 4), jnp.int32),
            pltpu.VMEM((B, 4), jnp.int32),
            pltpu.VMEM((B,), jnp.int32),
            pltpu.VMEM((B,), jnp.int32),
            pltpu.VMEM((B, 16), jnp.float32),
            pltpu.VMEM((B, 16), jnp.float32),
            pltpu.VMEM((16, ACC_R), jnp.float32),
            pltpu.SemaphoreType.DMA,
            pltpu.SemaphoreType.DMA,
            pltpu.SemaphoreType.DMA,
            pltpu.SemaphoreType.DMA,
        ],
    )
    def k(x_hbm, meta_hbm, a_out, meta_v0, meta_v1, colg_v0, colg_v1,
          xbuf0, xbuf1, acc_t, sm0, sm1, sg0, sg1):
        c = lax.axis_index("c")
        s = lax.axis_index("s")
        row_base = c * HALF
        zero16 = jnp.zeros((16,), jnp.float32)
        iota = lax.iota(jnp.int32, 16)

        def fire_meta(b, mv, sem):
            pltpu.async_copy(meta_hbm.at[pl.ds(b * B, B), :], mv, sem)

        def wait_meta(mv, sem):
            pltpu.make_async_copy(meta_hbm.at[pl.ds(0, B), :], mv, sem).wait()

        def prep(mv, cv):
            def prep_body(kk, carry):
                eidx = kk * 16 + iota
                cols16 = plsc.load_gather(mv, [eidx, jnp.full((16,), 1, jnp.int32)])
                cv[pl.ds(kk * 16, 16)] = cols16 * 16 + s
                return carry
            lax.fori_loop(0, B // 16, prep_body, 0)

        def fire_gathers(cv, xb, sem):
            for q in range(B // GB):
                pltpu.async_copy(x_hbm.at[cv.at[pl.ds(q * GB, GB)]],
                                 xb.at[pl.ds(q * GB, GB)], sem)

        def wait_gathers(cv, xb, sem):
            for q in range(B // GB):
                pltpu.make_async_copy(x_hbm.at[cv.at[pl.ds(q * GB, GB)]],
                                      xb.at[pl.ds(q * GB, GB)], sem).wait()

        def compute(mv, xb):
            def grp_body(kk, carry):
                eidx = kk * 16 + iota
                rows16 = plsc.load_gather(mv, [eidx, jnp.full((16,), 0, jnp.int32)])
                vrr = plsc.bitcast(
                    plsc.load_gather(mv, [eidx, jnp.full((16,), 2, jnp.int32)]),
                    jnp.float32)
                vii = plsc.bitcast(
                    plsc.load_gather(mv, [eidx, jnp.full((16,), 3, jnp.int32)]),
                    jnp.float32)
                local = rows16 - row_base
                ok = (local >= 0) & (local < HALF)
                lr = jnp.where(ok, local, DUMMY)
                for m in range(8):
                    xr = plsc.load_gather(xb, [eidx, jnp.full((16,), 2 * m, jnp.int32)])
                    xi = plsc.load_gather(xb, [eidx, jnp.full((16,), 2 * m + 1, jnp.int32)])
                    orv = vrr * xr - vii * xi
                    oiv = vii * xr + vrr * xi
                    plsc.addupdate_scatter(
                        acc_t, [jnp.full((16,), 2 * m, jnp.int32), lr], orv)
                    plsc.addupdate_scatter(
                        acc_t, [jnp.full((16,), 2 * m + 1, jnp.int32), lr], oiv)
                return carry
            lax.fori_loop(0, B // 16, grp_body, 0)

        fire_meta(0, meta_v0, sm0)
        fire_meta(1, meta_v1, sm1)

        def zrow(r, carry):
            for ch in range(16):
                acc_t[ch, pl.ds(r * 16, 16)] = zero16
            return carry
        lax.fori_loop(0, ACC_R // 16, zrow, 0)

        wait_meta(meta_v0, sm0)
        prep(meta_v0, colg_v0)
        fire_gathers(colg_v0, xbuf0, sg0)

        def pair_body(t, carry):
            b0 = 2 * t
            wait_meta(meta_v1, sm1)
            prep(meta_v1, colg_v1)
            fire_gathers(colg_v1, xbuf1, sg1)

            wait_gathers(colg_v0, xbuf0, sg0)
            compute(meta_v0, xbuf0)

            @pl.when(b0 + 2 < NB)
            def _():
                fire_meta(b0 + 2, meta_v0, sm0)

            wait_gathers(colg_v1, xbuf1, sg1)
            compute(meta_v1, xbuf1)

            @pl.when(b0 + 3 < NB)
            def _():
                fire_meta(b0 + 3, meta_v1, sm1)

            @pl.when(b0 + 2 < NB)
            def _():
                wait_meta(meta_v0, sm0)
                prep(meta_v0, colg_v0)
                fire_gathers(colg_v0, xbuf0, sg0)
            return carry
        lax.fori_loop(0, NB // 2, pair_body, 0)

        pltpu.sync_copy(acc_t, a_out.at[c, pl.ds(16 * s, 16), :])

    return k(x_flat, meta)


def _tc_body(a_ref, w_ref, xr_ref, xi_ref, or_ref, oi_ref):
    a = a_ref[0]
    res = lax.dot_general(a, w_ref[...], (((0,), (0,)), ((), ())),
                          preferred_element_type=jnp.float32)
    or_ref[...] = res[:HALF, :C] + xr_ref[...]
    oi_ref[...] = res[:HALF, C:] + xi_ref[...]


def _tc_matmul(a_full, x_real, x_imag, w_big):
    return pl.pallas_call(
        _tc_body,
        grid=(2,),
        in_specs=[
            pl.BlockSpec((1, C2, ACC_R), lambda i: (i, 0, 0)),
            pl.BlockSpec((C2, C2), lambda i: (0, 0)),
            pl.BlockSpec((HALF, C), lambda i: (i, 0)),
            pl.BlockSpec((HALF, C), lambda i: (i, 0)),
        ],
        out_specs=[
            pl.BlockSpec((HALF, C), lambda i: (i, 0)),
            pl.BlockSpec((HALF, C), lambda i: (i, 0)),
        ],
        out_shape=[
            jax.ShapeDtypeStruct((N, C), jnp.float32),
            jax.ShapeDtypeStruct((N, C), jnp.float32),
        ],
    )(a_full, w_big, x_real, x_imag)


@jax.jit
def kernel(X_real, X_imag, edge_index, L_real_vals, L_imag_vals, weight):
    x_flat = jnp.stack([X_real, X_imag], axis=2).reshape(N * 16, 16)
    meta = jnp.stack(
        [edge_index[0], edge_index[1],
         jax.lax.bitcast_convert_type(L_real_vals, jnp.int32),
         jax.lax.bitcast_convert_type(L_imag_vals, jnp.int32)], axis=1)
    ch = jnp.arange(C)
    u_r = (ch // 8) * 16 + (ch % 8) * 2
    w_big = jnp.zeros((C2, C2), jnp.float32)
    w_big = w_big.at[u_r, :C].set(weight).at[u_r + 1, C:].set(weight)

    a_full = _sc_spmm(x_flat, meta)
    return _tc_matmul(a_full, X_real, X_imag, w_big)


# speedup vs baseline: 1.5377x; 1.2168x over previous
import functools

import jax
import jax.numpy as jnp
from jax import lax
from jax.experimental import pallas as pl
from jax.experimental.pallas import tpu as pltpu
from jax.experimental.pallas import tpu_sc as plsc

N = 10000
C = 128
E = 320000
HALF = N // 2
ACC_R = 5200
DUMMY = HALF
B = 640
NB = E // B
GB = 128
RB = 200
C2 = 2 * C


def _sc_spmm(x_flat, meta, n_tiles_arg=None):
    mesh = plsc.VectorSubcoreMesh(core_axis_name="c", subcore_axis_name="s")

    @functools.partial(
        pl.kernel,
        mesh=mesh,
        compiler_params=pltpu.CompilerParams(
            needs_layout_passes=False, use_tc_tiling_on_sc=False,
            disable_bounds_checks=True),
        out_type=jax.ShapeDtypeStruct((2, C2, ACC_R), jnp.float32),
        scratch_types=[
            pltpu.VMEM((B, 4), jnp.int32),
            pltpu.VMEM((B, 4), jnp.int32),
            pltpu.VMEM((B,), jnp.int32),
            pltpu.VMEM((B,), jnp.int32),
            pltpu.VMEM((B, 16), jnp.float32),
            pltpu.VMEM((B, 16), jnp.float32),
            pltpu.VMEM((16, ACC_R), jnp.float32),
            pltpu.SemaphoreType.DMA,
            pltpu.SemaphoreType.DMA,
            pltpu.SemaphoreType.DMA,
            pltpu.SemaphoreType.DMA,
        ],
    )
    def k(x_hbm, meta_hbm, a_out, meta_v0, meta_v1, colg_v0, colg_v1,
          xbuf0, xbuf1, acc_t, sm0, sm1, sg0, sg1):
        c = lax.axis_index("c")
        s = lax.axis_index("s")
        row_base = c * HALF
        zero16 = jnp.zeros((16,), jnp.float32)
        iota = lax.iota(jnp.int32, 16)

        def fire_meta(b, mv, sem):
            pltpu.async_copy(meta_hbm.at[pl.ds(b * B, B), :], mv, sem)

        def wait_meta(mv, sem):
            pltpu.make_async_copy(meta_hbm.at[pl.ds(0, B), :], mv, sem).wait()

        def prep(mv, cv):
            @plsc.parallel_loop(0, B // 16, unroll=2)
            def _(kk):
                eidx = kk * 16 + iota
                cols16 = plsc.load_gather(mv, [eidx, jnp.full((16,), 1, jnp.int32)])
                cv[pl.ds(kk * 16, 16)] = cols16 * 16 + s

        def fire_gathers(cv, xb, sem):
            for q in range(B // GB):
                pltpu.async_copy(x_hbm.at[cv.at[pl.ds(q * GB, GB)]],
                                 xb.at[pl.ds(q * GB, GB)], sem)

        def wait_gathers(cv, xb, sem):
            for q in range(B // GB):
                pltpu.make_async_copy(x_hbm.at[cv.at[pl.ds(q * GB, GB)]],
                                      xb.at[pl.ds(q * GB, GB)], sem).wait()

        def compute(mv, xb):
            @plsc.parallel_loop(0, B // 16, unroll=2)
            def grp_body(kk):
                eidx = kk * 16 + iota
                rows16 = plsc.load_gather(mv, [eidx, jnp.full((16,), 0, jnp.int32)])
                vrr = plsc.bitcast(
                    plsc.load_gather(mv, [eidx, jnp.full((16,), 2, jnp.int32)]),
                    jnp.float32)
                vii = plsc.bitcast(
                    plsc.load_gather(mv, [eidx, jnp.full((16,), 3, jnp.int32)]),
                    jnp.float32)
                local = rows16 - row_base
                ok = (local >= 0) & (local < HALF)
                lr = jnp.where(ok, local, DUMMY)
                for m in range(8):
                    xr = plsc.load_gather(xb, [eidx, jnp.full((16,), 2 * m, jnp.int32)])
                    xi = plsc.load_gather(xb, [eidx, jnp.full((16,), 2 * m + 1, jnp.int32)])
                    orv = vrr * xr - vii * xi
                    oiv = vii * xr + vrr * xi
                    plsc.addupdate_scatter(
                        acc_t, [jnp.full((16,), 2 * m, jnp.int32), lr], orv)
                    plsc.addupdate_scatter(
                        acc_t, [jnp.full((16,), 2 * m + 1, jnp.int32), lr], oiv)

        fire_meta(0, meta_v0, sm0)
        fire_meta(1, meta_v1, sm1)

        def zrow(r, carry):
            for ch in range(16):
                acc_t[ch, pl.ds(r * 16, 16)] = zero16
            return carry
        lax.fori_loop(0, ACC_R // 16, zrow, 0)

        wait_meta(meta_v0, sm0)
        prep(meta_v0, colg_v0)
        fire_gathers(colg_v0, xbuf0, sg0)

        def pair_body(t, carry):
            b0 = 2 * t
            wait_meta(meta_v1, sm1)
            prep(meta_v1, colg_v1)
            fire_gathers(colg_v1, xbuf1, sg1)

            wait_gathers(colg_v0, xbuf0, sg0)
            compute(meta_v0, xbuf0)

            @pl.when(b0 + 2 < NB)
            def _():
                fire_meta(b0 + 2, meta_v0, sm0)

            wait_gathers(colg_v1, xbuf1, sg1)
            compute(meta_v1, xbuf1)

            @pl.when(b0 + 3 < NB)
            def _():
                fire_meta(b0 + 3, meta_v1, sm1)

            @pl.when(b0 + 2 < NB)
            def _():
                wait_meta(meta_v0, sm0)
                prep(meta_v0, colg_v0)
                fire_gathers(colg_v0, xbuf0, sg0)
            return carry
        lax.fori_loop(0, NB // 2, pair_body, 0)

        pltpu.sync_copy(acc_t, a_out.at[c, pl.ds(16 * s, 16), :])

    return k(x_flat, meta)


def _tc_body(a_ref, w_ref, xr_ref, xi_ref, or_ref, oi_ref):
    a = a_ref[0]
    res = lax.dot_general(a, w_ref[...], (((0,), (0,)), ((), ())),
                          preferred_element_type=jnp.float32)
    or_ref[...] = res[:HALF, :C] + xr_ref[...]
    oi_ref[...] = res[:HALF, C:] + xi_ref[...]


def _tc_matmul(a_full, x_real, x_imag, w_big):
    return pl.pallas_call(
        _tc_body,
        grid=(2,),
        in_specs=[
            pl.BlockSpec((1, C2, ACC_R), lambda i: (i, 0, 0)),
            pl.BlockSpec((C2, C2), lambda i: (0, 0)),
            pl.BlockSpec((HALF, C), lambda i: (i, 0)),
            pl.BlockSpec((HALF, C), lambda i: (i, 0)),
        ],
        out_specs=[
            pl.BlockSpec((HALF, C), lambda i: (i, 0)),
            pl.BlockSpec((HALF, C), lambda i: (i, 0)),
        ],
        out_shape=[
            jax.ShapeDtypeStruct((N, C), jnp.float32),
            jax.ShapeDtypeStruct((N, C), jnp.float32),
        ],
    )(a_full, w_big, x_real, x_imag)


@jax.jit
def kernel(X_real, X_imag, edge_index, L_real_vals, L_imag_vals, weight):
    x_flat = jnp.stack([X_real, X_imag], axis=2).reshape(N * 16, 16)
    meta = jnp.stack(
        [edge_index[0], edge_index[1],
         jax.lax.bitcast_convert_type(L_real_vals, jnp.int32),
         jax.lax.bitcast_convert_type(L_imag_vals, jnp.int32)], axis=1)
    ch = jnp.arange(C)
    u_r = (ch // 8) * 16 + (ch % 8) * 2
    w_big = jnp.zeros((C2, C2), jnp.float32)
    w_big = w_big.at[u_r, :C].set(weight).at[u_r + 1, C:].set(weight)

    a_full = _sc_spmm(x_flat, meta)
    return _tc_matmul(a_full, X_real, X_imag, w_big)
